# baseline (speedup 1.0000x reference)
import functools

import jax
import jax.numpy as jnp
from jax import lax
from jax.experimental import pallas as pl
from jax.experimental.pallas import tpu as pltpu
from jax.experimental.pallas import tpu_sc as plsc

DIM = 33
TBL = DIM * DIM * DIM
TBL_PAD = 35944
NLUT = 3 * TBL_PAD
BINSIZE = 1.000001 / (DIM - 1)
INV_BIN = float(1.0 / BINSIZE)


def _pack_lut(lut):
  flat = lut.reshape(3, TBL)
  u = lax.bitcast_convert_type(flat, jnp.uint32).astype(jnp.int64)
  u = jnp.pad(u, ((0, 0), (0, 1)))
  u1 = u[:, 1:]
  rb1 = (u1 + 0x7FFF + ((u1 >> 16) & 1)) >> 16
  rb0 = jnp.clip((u[:, :-1] - rb1 + 0x8000) >> 16, 0, 0xFFFF)
  word = ((rb0 << 16) | rb1).astype(jnp.uint32)
  word = jnp.pad(word, ((0, 0), (0, TBL_PAD - TBL)))
  return lax.bitcast_convert_type(word, jnp.int32).reshape(-1)


@functools.lru_cache(maxsize=None)
def _build(n_batch, pixels):
  info = plsc.get_sparse_core_info()
  NC, NS, L = info.num_cores, info.num_subcores, info.num_lanes
  NW = NC * NS
  ppw = pixels // NW
  C = 1024
  steps = ppw // C
  T = n_batch * steps
  chan_stride = pixels
  batch_stride = 3 * pixels

  mesh = plsc.VectorSubcoreMesh(core_axis_name="c", subcore_axis_name="s")

  buf = lambda: pltpu.VMEM((C,), jnp.float32)
  HI = jnp.int32(-65536)

  @functools.partial(
      pl.kernel,
      mesh=mesh,
      compiler_params=pltpu.CompilerParams(needs_layout_passes=False),
      out_type=jax.ShapeDtypeStruct((n_batch * 3 * pixels,), jnp.float32),
      scratch_types=[
          pltpu.VMEM((TBL_PAD,), jnp.int32),
          pltpu.VMEM((TBL_PAD,), jnp.int32),
          pltpu.VMEM((TBL_PAD,), jnp.int32),
          buf(), buf(), buf(), buf(), buf(), buf(),
          buf(), buf(), buf(), buf(), buf(), buf(),
          pltpu.SemaphoreType.DMA, pltpu.SemaphoreType.DMA,
          pltpu.SemaphoreType.DMA, pltpu.SemaphoreType.DMA,
      ],
  )
  def sc_kernel(lut_hbm, x_hbm, out_hbm, lut0, lut1, lut2,
                rvA, gvA, bvA, rvB, gvB, bvB,
                orA, ogA, obA, orB, ogB, obB,
                siA, siB, soA, soB):
    wid = lax.axis_index("s") * NC + lax.axis_index("c")
    pltpu.sync_copy(lut_hbm.at[pl.ds(0, TBL_PAD)], lut0)
    pltpu.sync_copy(lut_hbm.at[pl.ds(TBL_PAD, TBL_PAD)], lut1)
    pltpu.sync_copy(lut_hbm.at[pl.ds(2 * TBL_PAD, TBL_PAD)], lut2)
    base0 = wid * ppw

    def t_start(t):
      b = t // steps
      s = t - b * steps
      return b * batch_stride + base0 + s * C

    def issue_in(t, rv, gv, bv, sem):
      start = t_start(t)
      pltpu.async_copy(x_hbm.at[pl.ds(start, C)], rv, sem)
      pltpu.async_copy(x_hbm.at[pl.ds(start + chan_stride, C)], gv, sem)
      pltpu.async_copy(x_hbm.at[pl.ds(start + 2 * chan_stride, C)], bv, sem)

    def issue_out(t, orv, ogv, obv, sem):
      start = t_start(t)
      pltpu.async_copy(orv, out_hbm.at[pl.ds(start, C)], sem)
      pltpu.async_copy(ogv, out_hbm.at[pl.ds(start + chan_stride, C)], sem)
      pltpu.async_copy(obv, out_hbm.at[pl.ds(start + 2 * chan_stride, C)], sem)

    def drain3(sem, dst):
      for _ in range(3):
        pltpu.make_async_copy(x_hbm.at[pl.ds(0, C)], dst, sem).wait()

    def compute(rv, gv, bv, orv, ogv, obv):
      def vec(i, c2):
        off = i * L
        rs = rv[pl.ds(off, L)] * INV_BIN
        gs = gv[pl.ds(off, L)] * INV_BIN
        bs = bv[pl.ds(off, L)] * INV_BIN
        ri = rs.astype(jnp.int32)
        gi = gs.astype(jnp.int32)
        bi = bs.astype(jnp.int32)
        rd = rs - ri.astype(jnp.float32)
        gd = gs - gi.astype(jnp.float32)
        bd = bs - bi.astype(jnp.float32)
        gd1 = 1.0 - gd
        bd1 = 1.0 - bd
        w = (gd1 * bd1, gd * bd1, gd1 * bd, gd * bd)
        base = ri + gi * DIM + bi * (DIM * DIM)
        offs = (0, DIM, DIM * DIM, DIM * DIM + DIM)
        idx = [base + o if o else base for o in offs]
        pk = [plsc.load_gather(tbl, [ix])
              for tbl in (lut0, lut1, lut2) for ix in idx]
        lo = [plsc.bitcast(q, jnp.float32) for q in pk]
        hi = [plsc.bitcast(q << 16, jnp.float32) for q in pk]
        res = []
        for c in range(3):
          j = 4 * c
          a = ((w[0] * lo[j] + w[1] * lo[j + 1])
               + (w[2] * lo[j + 2] + w[3] * lo[j + 3]))
          b = ((w[0] * hi[j] + w[1] * hi[j + 1])
               + (w[2] * hi[j + 2] + w[3] * hi[j + 3]))
          res.append(a + rd * (b - a))
        orv[pl.ds(off, L)] = res[0]
        ogv[pl.ds(off, L)] = res[1]
        obv[pl.ds(off, L)] = res[2]
        return c2

      lax.fori_loop(0, C // L, vec, 0)

    issue_in(0, rvA, gvA, bvA, siA)
    issue_in(1, rvB, gvB, bvB, siB)

    def body(k, carry):
      tA = 2 * k
      tB = 2 * k + 1

      drain3(siA, rvA)
      @pl.when(k > 0)
      def _():
        drain3(soA, orA)
      compute(rvA, gvA, bvA, orA, ogA, obA)
      issue_out(tA, orA, ogA, obA, soA)
      @pl.when(tA + 2 < T)
      def _():
        issue_in(tA + 2, rvA, gvA, bvA, siA)

      drain3(siB, rvB)
      @pl.when(k > 0)
      def _():
        drain3(soB, orB)
      compute(rvB, gvB, bvB, orB, ogB, obB)
      issue_out(tB, orB, ogB, obB, soB)
      @pl.when(tB + 2 < T)
      def _():
        issue_in(tB + 2, rvB, gvB, bvB, siB)
      return carry

    lax.fori_loop(0, T // 2, body, 0)
    drain3(soA, orA)
    drain3(soB, orB)

  return sc_kernel


def kernel(lut_count, lut, x):
  n_batch = x.shape[0]
  pixels = x.shape[2] * x.shape[3]
  fn = _build(n_batch, pixels)
  out = fn(_pack_lut(lut), x.reshape(-1))
  return (lut, out.reshape(x.shape))


# speedup vs baseline: 1.0842x; 1.0842x over previous
import functools

import jax
import jax.numpy as jnp
from jax import lax
from jax.experimental import pallas as pl
from jax.experimental.pallas import tpu as pltpu
from jax.experimental.pallas import tpu_sc as plsc

DIM = 33
TBL = DIM * DIM * DIM
TBL_PAD = 35944
NLUT = 3 * TBL_PAD
BINSIZE = 1.000001 / (DIM - 1)
INV_BIN = float(1.0 / BINSIZE)


def _pack_lut(lut):
  flat = lut.reshape(3, TBL)
  u = lax.bitcast_convert_type(flat, jnp.uint32).astype(jnp.int64)
  u = jnp.pad(u, ((0, 0), (0, 1)))
  u1 = u[:, 1:]
  rb1 = (u1 + 0x7FFF + ((u1 >> 16) & 1)) >> 16
  rb0 = jnp.clip((u[:, :-1] - rb1 + 0x8000) >> 16, 0, 0xFFFF)
  word = ((rb0 << 16) | rb1).astype(jnp.uint32)
  word = jnp.pad(word, ((0, 0), (0, TBL_PAD - TBL)))
  return lax.bitcast_convert_type(word, jnp.int32).reshape(-1)


@functools.lru_cache(maxsize=None)
def _build(n_batch, pixels):
  info = plsc.get_sparse_core_info()
  NC, NS, L = info.num_cores, info.num_subcores, info.num_lanes
  NW = NC * NS
  ppw = pixels // NW
  C = 1024
  steps = ppw // C
  T = n_batch * steps
  chan_stride = pixels
  batch_stride = 3 * pixels

  mesh = plsc.VectorSubcoreMesh(core_axis_name="c", subcore_axis_name="s")

  buf = lambda: pltpu.VMEM((C,), jnp.float32)
  HI = jnp.int32(-65536)

  @functools.partial(
      pl.kernel,
      mesh=mesh,
      compiler_params=pltpu.CompilerParams(needs_layout_passes=False),
      out_type=jax.ShapeDtypeStruct((n_batch * 3 * pixels,), jnp.float32),
      scratch_types=[
          pltpu.VMEM((NLUT,), jnp.int32),
          buf(), buf(), buf(), buf(), buf(), buf(),
          buf(), buf(), buf(), buf(), buf(), buf(),
          pltpu.SemaphoreType.DMA, pltpu.SemaphoreType.DMA,
          pltpu.SemaphoreType.DMA, pltpu.SemaphoreType.DMA,
      ],
  )
  def sc_kernel(lut_hbm, x_hbm, out_hbm, lut_v,
                rvA, gvA, bvA, rvB, gvB, bvB,
                orA, ogA, obA, orB, ogB, obB,
                siA, siB, soA, soB):
    wid = lax.axis_index("s") * NC + lax.axis_index("c")
    pltpu.sync_copy(lut_hbm, lut_v)
    base0 = wid * ppw

    def t_start(t):
      b = t // steps
      s = t - b * steps
      return b * batch_stride + base0 + s * C

    def issue_in(t, rv, gv, bv, sem):
      start = t_start(t)
      pltpu.async_copy(x_hbm.at[pl.ds(start, C)], rv, sem)
      pltpu.async_copy(x_hbm.at[pl.ds(start + chan_stride, C)], gv, sem)
      pltpu.async_copy(x_hbm.at[pl.ds(start + 2 * chan_stride, C)], bv, sem)

    def issue_out(t, orv, ogv, obv, sem):
      start = t_start(t)
      pltpu.async_copy(orv, out_hbm.at[pl.ds(start, C)], sem)
      pltpu.async_copy(ogv, out_hbm.at[pl.ds(start + chan_stride, C)], sem)
      pltpu.async_copy(obv, out_hbm.at[pl.ds(start + 2 * chan_stride, C)], sem)

    def drain3(sem, dst):
      for _ in range(3):
        pltpu.make_async_copy(x_hbm.at[pl.ds(0, C)], dst, sem).wait()

    def compute(rv, gv, bv, orv, ogv, obv):
      def vec(i, c2):
        off = i * L
        rs = rv[pl.ds(off, L)] * INV_BIN
        gs = gv[pl.ds(off, L)] * INV_BIN
        bs = bv[pl.ds(off, L)] * INV_BIN
        ri = rs.astype(jnp.int32)
        gi = gs.astype(jnp.int32)
        bi = bs.astype(jnp.int32)
        rd = rs - ri.astype(jnp.float32)
        gd = gs - gi.astype(jnp.float32)
        bd = bs - bi.astype(jnp.float32)
        gd1 = 1.0 - gd
        bd1 = 1.0 - bd
        w = (gd1 * bd1, gd * bd1, gd1 * bd, gd * bd)
        base = ri + gi * DIM + bi * (DIM * DIM)
        offs = (0, DIM, DIM * DIM, DIM * DIM + DIM)
        pk = [plsc.load_gather(lut_v, [base + (c * TBL_PAD + o)])
              for c in range(3) for o in offs]
        lo = [plsc.bitcast(q, jnp.float32) for q in pk]
        hi = [plsc.bitcast(q << 16, jnp.float32) for q in pk]
        res = []
        for c in range(3):
          j = 4 * c
          a = ((w[0] * lo[j] + w[1] * lo[j + 1])
               + (w[2] * lo[j + 2] + w[3] * lo[j + 3]))
          b = ((w[0] * hi[j] + w[1] * hi[j + 1])
               + (w[2] * hi[j + 2] + w[3] * hi[j + 3]))
          res.append(a + rd * (b - a))
        orv[pl.ds(off, L)] = res[0]
        ogv[pl.ds(off, L)] = res[1]
        obv[pl.ds(off, L)] = res[2]
        return c2

      lax.fori_loop(0, C // L, vec, 0)

    issue_in(0, rvA, gvA, bvA, siA)
    issue_in(1, rvB, gvB, bvB, siB)

    def body(k, carry):
      tA = 2 * k
      tB = 2 * k + 1

      drain3(siA, rvA)
      @pl.when(k > 0)
      def _():
        drain3(soA, orA)
      compute(rvA, gvA, bvA, orA, ogA, obA)
      issue_out(tA, orA, ogA, obA, soA)
      @pl.when(tA + 2 < T)
      def _():
        issue_in(tA + 2, rvA, gvA, bvA, siA)

      drain3(siB, rvB)
      @pl.when(k > 0)
      def _():
        drain3(soB, orB)
      compute(rvB, gvB, bvB, orB, ogB, obB)
      issue_out(tB, orB, ogB, obB, soB)
      @pl.when(tB + 2 < T)
      def _():
        issue_in(tB + 2, rvB, gvB, bvB, siB)
      return carry

    lax.fori_loop(0, T // 2, body, 0)
    drain3(soA, orA)
    drain3(soB, orB)

  return sc_kernel


def kernel(lut_count, lut, x):
  n_batch = x.shape[0]
  pixels = x.shape[2] * x.shape[3]
  fn = _build(n_batch, pixels)
  out = fn(_pack_lut(lut), x.reshape(-1))
  return (lut, out.reshape(x.shape))
